# baseline (speedup 1.0000x reference)
import functools

import jax
import jax.numpy as jnp
from jax import lax
from jax.experimental import pallas as pl
from jax.experimental.pallas import tpu as pltpu
from jax.experimental.pallas import tpu_sc as plsc

_ROWS = 100000
_EMBED = 64


_TC_BLK = 10000


def _tc_copy_kernel(x_in, x_out):
    x_out[...] = x_in[...]


def _tc_copy(x):
    spec = pl.BlockSpec((_TC_BLK, _EMBED), lambda n: (n, 0))
    return pl.pallas_call(
        _tc_copy_kernel,
        grid=(_ROWS // _TC_BLK,),
        out_shape=jax.ShapeDtypeStruct(x.shape, x.dtype),
        in_specs=[spec],
        out_specs=spec,
    )(x)


_NW = 32
_CHUNK = 400
_NCHUNK = _ROWS // _CHUNK
_ROUNDS = -(-_NCHUNK // _NW)


@functools.partial(
    pl.kernel,
    out_type=jax.ShapeDtypeStruct((_ROWS, _EMBED), jnp.float32),
    mesh=plsc.VectorSubcoreMesh(core_axis_name="c", subcore_axis_name="s"),
    scratch_types=[
        pltpu.VMEM((_CHUNK, _EMBED), jnp.float32),
        pltpu.VMEM((_CHUNK, _EMBED), jnp.float32),
        pltpu.SemaphoreType.DMA,
        pltpu.SemaphoreType.DMA,
    ],
)
def _sc_copy(x_in, x_out, buf0, buf1, sem0, sem1):
    wid = lax.axis_index("s") * 2 + lax.axis_index("c")
    bufs = (buf0, buf1)
    sems = (sem0, sem1)

    for r in range(_ROUNDS):
        c = r * _NW + wid
        off = c * _CHUNK
        j = r % 2

        def _round(off=off, j=j):
            sl = pl.ds(off, _CHUNK)
            g = pltpu.async_copy(x_in.at[sl], bufs[j], sems[j])
            g.wait()
            s = pltpu.async_copy(bufs[j], x_out.at[sl], sems[j])
            s.wait()

        if (r + 1) * _NW <= _NCHUNK:
            _round()
        else:
            pl.when(c < _NCHUNK)(_round)


def kernel(user_weight, item_weight):
    return _tc_copy(user_weight), _sc_copy(item_weight)
